# baseline (speedup 1.0000x reference)
import functools

import jax
import jax.numpy as jnp
from jax import lax
from jax.experimental import pallas as pl
from jax.experimental.pallas import tpu as pltpu
from jax.experimental.pallas import tpu_sc as plsc

_NC, _NS = 2, 16


def _sc_gather2(idx_flat, table_c, table_t, out_rows, kc, kt, chunk,
                chunks_per_worker):
    n_rows = idx_flat.shape[0]
    d = table_c.shape[1]
    per_worker = chunks_per_worker * chunk
    assert n_rows == per_worker * _NC * _NS
    regions_per_batch = (kc + kt) // per_worker
    ctx_regions = kc // per_worker
    mesh = plsc.VectorSubcoreMesh(core_axis_name="c", subcore_axis_name="s")

    @functools.partial(
        pl.kernel,
        mesh=mesh,
        out_type=jax.ShapeDtypeStruct((out_rows, d), jnp.float32),
        scratch_types=[
            pltpu.VMEM((per_worker,), jnp.int32),
            pltpu.VMEM((chunk, d), jnp.float32),
            pltpu.SemaphoreType.DMA,
        ],
    )
    def gather_k(idx_hbm, tc_hbm, tt_hbm, out_hbm, idx_v, rows_v, sem):
        wid = lax.axis_index("s") * _NC + lax.axis_index("c")
        base = wid * per_worker
        pltpu.sync_copy(idx_hbm.at[pl.ds(base, per_worker)], idx_v)
        is_ctx = lax.rem(wid, regions_per_batch) < ctx_regions

        @pl.when(is_ctx)
        def _():
            for j in range(chunks_per_worker):
                pltpu.async_copy(
                    tc_hbm.at[idx_v.at[pl.ds(j * chunk, chunk)]], rows_v, sem
                ).wait()
                pltpu.sync_copy(rows_v, out_hbm.at[pl.ds(base + j * chunk, chunk)])

        @pl.when(jnp.logical_not(is_ctx))
        def _():
            for j in range(chunks_per_worker):
                pltpu.async_copy(
                    tt_hbm.at[idx_v.at[pl.ds(j * chunk, chunk)]], rows_v, sem
                ).wait()
                pltpu.sync_copy(rows_v, out_hbm.at[pl.ds(base + j * chunk, chunk)])

    return gather_k(idx_flat, table_c, table_t)


def _tc_combine_a(hidden_states, W, b2, pos_all, nb, n_ctx_blocks, rb):
    B, Kc, E = hidden_states.shape
    D = W.shape[1]
    K_total = pos_all.shape[1]

    def body(hs_ref, w_ref, b_ref, pos_ref, out_ref):
        acc = jax.lax.dot_general(
            hs_ref[0].astype(jnp.bfloat16), w_ref[...].astype(jnp.bfloat16),
            (((1,), (0,)), ((), ())),
            preferred_element_type=jnp.float32,
        )
        out_ref[0] = acc + b_ref[...] + pos_ref[0]

    return pl.pallas_call(
        body,
        grid=(nb, n_ctx_blocks),
        in_specs=[
            pl.BlockSpec((1, rb, E), lambda i, r: (i, r, 0)),
            pl.BlockSpec((E, D), lambda i, r: (0, 0)),
            pl.BlockSpec((1, D), lambda i, r: (0, 0)),
            pl.BlockSpec((1, rb, D), lambda i, r: (i, r, 0)),
        ],
        out_specs=pl.BlockSpec((1, rb, D), lambda i, r: (i, r, 0)),
        out_shape=jax.ShapeDtypeStruct((B, K_total, D), jnp.float32),
        input_output_aliases={3: 0},
    )(hidden_states, W, b2, pos_all)


def _tc_combine_b(hidden_states, W, b2, y, x, b_off, rb):
    B, Kc, E = hidden_states.shape
    D = W.shape[1]
    nb = B - b_off
    K_total = x.shape[1]
    n_ctx_blocks = Kc // rb
    n_blocks = K_total // rb

    def body(hs_ref, w_ref, b_ref, y_ref, x_ref, out_ref):
        r = pl.program_id(1)

        @pl.when(r < n_ctx_blocks)
        def _():
            acc = jax.lax.dot_general(
                hs_ref[0].astype(jnp.bfloat16), w_ref[...].astype(jnp.bfloat16),
                (((1,), (0,)), ((), ())),
                preferred_element_type=jnp.float32,
            )
            out_ref[0] = acc + b_ref[...] + y_ref[0]

        @pl.when(r >= n_ctx_blocks)
        def _():
            out_ref[0] = y_ref[0]

    return pl.pallas_call(
        body,
        grid=(nb, n_blocks),
        in_specs=[
            pl.BlockSpec((1, rb, E),
                         lambda i, r: (i + b_off, jnp.minimum(r, n_ctx_blocks - 1), 0)),
            pl.BlockSpec((E, D), lambda i, r: (0, 0)),
            pl.BlockSpec((1, D), lambda i, r: (0, 0)),
            pl.BlockSpec((1, rb, D), lambda i, r: (i, r, 0)),
            pl.BlockSpec((1, rb, D), lambda i, r: (i + b_off, r, 0)),
        ],
        out_specs=pl.BlockSpec((1, rb, D), lambda i, r: (i + b_off, r, 0)),
        out_shape=jax.ShapeDtypeStruct((B, K_total, D), jnp.float32),
        input_output_aliases={4: 0},
    )(hidden_states, W, b2, y, x)


def kernel(hidden_states, context_mask, target_mask, mask_index, W, b, mask_token, pos_embed):
    B, Kc, E = hidden_states.shape
    Kt = target_mask.shape[1]
    D = W.shape[1]
    K_total = Kc + Kt
    half = B // 2

    masks = jnp.concatenate([context_mask, target_mask], axis=1)
    table_t = pos_embed + mask_token[0]

    chunk = 96
    half_rows = half * K_total
    chunks_per_worker = half_rows // (_NC * _NS * chunk)

    idx_a = masks[:half].reshape(half_rows)
    idx_b = masks[half:].reshape(half_rows)

    x = _sc_gather2(idx_a, pos_embed, table_t, B * K_total, Kc, Kt, chunk,
                    chunks_per_worker).reshape(B, K_total, D)
    y = _sc_gather2(idx_b, pos_embed, table_t, half_rows, Kc, Kt, chunk,
                    chunks_per_worker).reshape(half, K_total, D)

    b2 = b.reshape(1, D)
    rb = 576
    x = _tc_combine_a(hidden_states, W, b2, x, half, Kc // rb, rb)
    embeddings = _tc_combine_b(hidden_states, W, b2, y, x, half, rb)
    return (embeddings, masks)


# speedup vs baseline: 1.0671x; 1.0671x over previous
import functools

import jax
import jax.numpy as jnp
from jax import lax
from jax.experimental import pallas as pl
from jax.experimental.pallas import tpu as pltpu
from jax.experimental.pallas import tpu_sc as plsc

_NC, _NS = 2, 16


def _sc_gather(idx_flat, table, out_rows, chunk, chunks_per_worker):
    n_rows = idx_flat.shape[0]
    d = table.shape[1]
    per_worker = chunks_per_worker * chunk
    assert n_rows == per_worker * _NC * _NS
    mesh = plsc.VectorSubcoreMesh(core_axis_name="c", subcore_axis_name="s")

    @functools.partial(
        pl.kernel,
        mesh=mesh,
        out_type=jax.ShapeDtypeStruct((out_rows, d), jnp.float32),
        scratch_types=[
            pltpu.VMEM((per_worker,), jnp.int32),
            pltpu.VMEM((chunk, d), jnp.float32),
            pltpu.SemaphoreType.DMA,
        ],
    )
    def gather_k(idx_hbm, table_hbm, out_hbm, idx_v, rows_v, sem):
        wid = lax.axis_index("s") * _NC + lax.axis_index("c")
        base = wid * per_worker
        pltpu.sync_copy(idx_hbm.at[pl.ds(base, per_worker)], idx_v)
        for j in range(chunks_per_worker):
            pltpu.async_copy(
                table_hbm.at[idx_v.at[pl.ds(j * chunk, chunk)]], rows_v, sem
            ).wait()
            pltpu.sync_copy(rows_v, out_hbm.at[pl.ds(base + j * chunk, chunk)])

    return gather_k(idx_flat, table)


def _tc_combine_a(hidden_states, W, b2, mt2, x, nb, rb):
    B, Kc, E = hidden_states.shape
    D = W.shape[1]
    K_total = x.shape[1]
    n_ctx_blocks = Kc // rb
    n_blocks = K_total // rb

    def body(hs_ref, w_ref, b_ref, mt_ref, pos_ref, out_ref):
        r = pl.program_id(1)

        @pl.when(r < n_ctx_blocks)
        def _():
            acc = jax.lax.dot_general(
                hs_ref[0].astype(jnp.bfloat16), w_ref[...].astype(jnp.bfloat16),
                (((1,), (0,)), ((), ())),
                preferred_element_type=jnp.float32,
            )
            out_ref[0] = acc + b_ref[...] + pos_ref[0]

        @pl.when(r >= n_ctx_blocks)
        def _():
            out_ref[0] = pos_ref[0] + mt_ref[...]

    return pl.pallas_call(
        body,
        grid=(nb, n_blocks),
        in_specs=[
            pl.BlockSpec((1, rb, E),
                         lambda i, r: (i, jnp.minimum(r, n_ctx_blocks - 1), 0)),
            pl.BlockSpec((E, D), lambda i, r: (0, 0)),
            pl.BlockSpec((1, D), lambda i, r: (0, 0)),
            pl.BlockSpec((1, D), lambda i, r: (0, 0)),
            pl.BlockSpec((1, rb, D), lambda i, r: (i, r, 0)),
        ],
        out_specs=pl.BlockSpec((1, rb, D), lambda i, r: (i, r, 0)),
        out_shape=jax.ShapeDtypeStruct((B, K_total, D), jnp.float32),
        input_output_aliases={4: 0},
    )(hidden_states, W, b2, mt2, x)


def _tc_combine_b(hidden_states, W, b2, mt2, y, x, b_off, rb):
    B, Kc, E = hidden_states.shape
    D = W.shape[1]
    K_total = x.shape[1]
    nb = B - b_off
    n_ctx_blocks = Kc // rb
    n_blocks = K_total // rb

    def body(hs_ref, w_ref, b_ref, mt_ref, y_ref, x_ref, out_ref):
        r = pl.program_id(1)

        @pl.when(r < n_ctx_blocks)
        def _():
            acc = jax.lax.dot_general(
                hs_ref[0].astype(jnp.bfloat16), w_ref[...].astype(jnp.bfloat16),
                (((1,), (0,)), ((), ())),
                preferred_element_type=jnp.float32,
            )
            out_ref[0] = acc + b_ref[...] + y_ref[0]

        @pl.when(r >= n_ctx_blocks)
        def _():
            out_ref[0] = y_ref[0] + mt_ref[...]

    return pl.pallas_call(
        body,
        grid=(nb, n_blocks),
        in_specs=[
            pl.BlockSpec((1, rb, E),
                         lambda i, r: (i + b_off, jnp.minimum(r, n_ctx_blocks - 1), 0)),
            pl.BlockSpec((E, D), lambda i, r: (0, 0)),
            pl.BlockSpec((1, D), lambda i, r: (0, 0)),
            pl.BlockSpec((1, D), lambda i, r: (0, 0)),
            pl.BlockSpec((1, rb, D), lambda i, r: (i, r, 0)),
            pl.BlockSpec(memory_space=pl.ANY),
        ],
        out_specs=pl.BlockSpec((1, rb, D), lambda i, r: (i + b_off, r, 0)),
        out_shape=jax.ShapeDtypeStruct((B, K_total, D), jnp.float32),
        input_output_aliases={5: 0},
    )(hidden_states, W, b2, mt2, y, x)


def kernel(hidden_states, context_mask, target_mask, mask_index, W, b, mask_token, pos_embed):
    B, Kc, E = hidden_states.shape
    Kt = target_mask.shape[1]
    D = W.shape[1]
    K_total = Kc + Kt
    half = B // 2

    masks = jnp.concatenate([context_mask, target_mask], axis=1)

    chunk = 96
    half_rows = half * K_total
    chunks_per_worker = half_rows // (_NC * _NS * chunk)

    idx_a = masks[:half].reshape(half_rows)
    idx_b = masks[half:].reshape(half_rows)

    x = _sc_gather(idx_a, pos_embed, B * K_total, chunk,
                   chunks_per_worker).reshape(B, K_total, D)
    y = _sc_gather(idx_b, pos_embed, half_rows, chunk,
                   chunks_per_worker).reshape(half, K_total, D)

    b2 = b.reshape(1, D)
    mt2 = mask_token.reshape(1, D)
    rb = 1152
    x = _tc_combine_a(hidden_states, W, b2, mt2, x, half, rb)
    embeddings = _tc_combine_b(hidden_states, W, b2, mt2, y, x, half, rb)
    return (embeddings, masks)
